# baseline (speedup 1.0000x reference)
import functools

import jax
import jax.numpy as jnp
from jax import lax
from jax.experimental import pallas as pl
from jax.experimental.pallas import tpu as pltpu
from jax.experimental.pallas import tpu_sc as plsc

GR = 256
LANES = 16
C = 128
NC = 2
NS = 16
NW = NC * NS


def _axis_terms(p):
    p = jnp.minimum(jnp.maximum(p, 0.0), 1.0 - 1e-6)
    gc = p * float(GR)
    gc = jnp.minimum(jnp.maximum(gc, 0.0), float(GR - 1))
    i0 = gc.astype(jnp.int32)
    i1 = jnp.minimum(i0 + 1, GR - 1)
    d = gc - i0.astype(jnp.float32)
    return i0, i1, d


def _make_sc_interp(npad, nchunks):
    mesh = plsc.VectorSubcoreMesh(core_axis_name="c", subcore_axis_name="s")
    niter = -(-nchunks // NW)

    @functools.partial(
        pl.kernel,
        mesh=mesh,
        out_type=jax.ShapeDtypeStruct((npad,), jnp.float32),
        scratch_types=[
            pltpu.VMEM((3, C), jnp.float32),
            pltpu.VMEM((8, C), jnp.int32),
            pltpu.VMEM((3, C), jnp.float32),
            pltpu.VMEM((8, C), jnp.float32),
            pltpu.VMEM((C,), jnp.float32),
            pltpu.SemaphoreType.DMA,
        ],
    )
    def sc_interp(xs_hbm, ys_hbm, zs_hbm, gridf_hbm, out_hbm, pts_v, idx_v,
                  wt_v, val_v, out_v, sem):
        wid = lax.axis_index("s") * NC + lax.axis_index("c")

        def iter_body(t, carry):
            chunk = wid + NW * t

            @pl.when(chunk < nchunks)
            def _():
                base = chunk * C
                pltpu.sync_copy(xs_hbm.at[pl.ds(base, C)], pts_v.at[0])
                pltpu.sync_copy(ys_hbm.at[pl.ds(base, C)], pts_v.at[1])
                pltpu.sync_copy(zs_hbm.at[pl.ds(base, C)], pts_v.at[2])

                def vec_body(j, carry2):
                    sb = j * LANES
                    dsv = pl.ds(sb, LANES)
                    px = pts_v[0, dsv]
                    py = pts_v[1, dsv]
                    pz = pts_v[2, dsv]
                    x0, x1, xd = _axis_terms(px)
                    y0, y1, yd = _axis_terms(py)
                    z0, z1, zd = _axis_terms(pz)
                    x0s = x0 << 16
                    x1s = x1 << 16
                    y0s = y0 << 8
                    y1s = y1 << 8
                    b00 = x0s + y0s
                    b01 = x0s + y1s
                    b10 = x1s + y0s
                    b11 = x1s + y1s
                    dsj = pl.ds(sb, LANES)
                    idx_v[0, dsj] = b00 + z0
                    idx_v[1, dsj] = b00 + z1
                    idx_v[2, dsj] = b01 + z0
                    idx_v[3, dsj] = b01 + z1
                    idx_v[4, dsj] = b10 + z0
                    idx_v[5, dsj] = b10 + z1
                    idx_v[6, dsj] = b11 + z0
                    idx_v[7, dsj] = b11 + z1
                    wt_v[0, dsj] = xd
                    wt_v[1, dsj] = yd
                    wt_v[2, dsj] = zd
                    return carry2

                lax.fori_loop(0, C // LANES, vec_body, 0)

                cps = [
                    pltpu.async_copy(gridf_hbm.at[idx_v.at[k]], val_v.at[k],
                                     sem)
                    for k in range(8)
                ]
                for cp in cps:
                    cp.wait()

                def mix_body(j, carry2):
                    dsj = pl.ds(j * LANES, LANES)
                    v000 = val_v[0, dsj]
                    v001 = val_v[1, dsj]
                    v010 = val_v[2, dsj]
                    v011 = val_v[3, dsj]
                    v100 = val_v[4, dsj]
                    v101 = val_v[5, dsj]
                    v110 = val_v[6, dsj]
                    v111 = val_v[7, dsj]
                    xd = wt_v[0, dsj]
                    yd = wt_v[1, dsj]
                    zd = wt_v[2, dsj]
                    c00 = v000 + (v100 - v000) * xd
                    c01 = v001 + (v101 - v001) * xd
                    c10 = v010 + (v110 - v010) * xd
                    c11 = v011 + (v111 - v011) * xd
                    c0 = c00 + (c10 - c00) * yd
                    c1 = c01 + (c11 - c01) * yd
                    out_v[dsj] = c0 + (c1 - c0) * zd
                    return carry2

                lax.fori_loop(0, C // LANES, mix_body, 0)
                pltpu.sync_copy(out_v, out_hbm.at[pl.ds(base, C)])

            return carry

        lax.fori_loop(0, niter, iter_body, 0)

    return sc_interp


def kernel(points, grid):
    npts = points.shape[0]
    nchunks = -(-npts // C)
    npad = nchunks * C
    pts = jnp.pad(points, ((0, npad - npts), (0, 0)))
    gridf = grid.reshape(-1)
    xs, ys, zs = pts[:, 0], pts[:, 1], pts[:, 2]
    out = _make_sc_interp(npad, nchunks)(xs, ys, zs, gridf)
    return out[:npts]


# speedup vs baseline: 1.8424x; 1.8424x over previous
import functools

import jax
import jax.numpy as jnp
from jax import lax
from jax.experimental import pallas as pl
from jax.experimental.pallas import tpu as pltpu
from jax.experimental.pallas import tpu_sc as plsc

GR = 256
LANES = 16
C = 1024
NC = 2
NS = 16
NW = NC * NS


def _axis_terms(p):
    p = jnp.minimum(jnp.maximum(p, 0.0), 1.0 - 1e-6)
    gc = p * float(GR)
    gc = jnp.minimum(jnp.maximum(gc, 0.0), float(GR - 1))
    i0 = gc.astype(jnp.int32)
    i1 = jnp.minimum(i0 + 1, GR - 1)
    d = gc - i0.astype(jnp.float32)
    return i0, i1, d


def _make_sc_interp(npad, nchunks):
    mesh = plsc.VectorSubcoreMesh(core_axis_name="c", subcore_axis_name="s")
    niter = -(-nchunks // NW)

    @functools.partial(
        pl.kernel,
        mesh=mesh,
        out_type=jax.ShapeDtypeStruct((npad,), jnp.float32),
        scratch_types=[
            pltpu.VMEM((3 * C,), jnp.float32),
            pltpu.VMEM((8 * C,), jnp.int32),
            pltpu.VMEM((3 * C,), jnp.float32),
            pltpu.VMEM((8 * C,), jnp.float32),
            pltpu.VMEM((C,), jnp.float32),
            pltpu.SemaphoreType.DMA,
        ],
    )
    def sc_interp(xyz_hbm, gridf_hbm, out_hbm, pts_v, idx_v, wt_v, val_v,
                  out_v, sem):
        wid = lax.axis_index("s") * NC + lax.axis_index("c")

        def iter_body(t, carry):
            chunk = wid + NW * t

            @pl.when(chunk < nchunks)
            def _():
                base = chunk * C
                pltpu.sync_copy(xyz_hbm.at[pl.ds(base * 3, 3 * C)], pts_v)

                def vec_body(j, carry2):
                    sb = j * LANES
                    px = pts_v[pl.ds(sb, LANES)]
                    py = pts_v[pl.ds(C + sb, LANES)]
                    pz = pts_v[pl.ds(2 * C + sb, LANES)]
                    x0, x1, xd = _axis_terms(px)
                    y0, y1, yd = _axis_terms(py)
                    z0, z1, zd = _axis_terms(pz)
                    x0s = x0 << 16
                    x1s = x1 << 16
                    y0s = y0 << 8
                    y1s = y1 << 8
                    b00 = x0s + y0s
                    b01 = x0s + y1s
                    b10 = x1s + y0s
                    b11 = x1s + y1s
                    idx_v[pl.ds(0 * C + sb, LANES)] = b00 + z0
                    idx_v[pl.ds(1 * C + sb, LANES)] = b00 + z1
                    idx_v[pl.ds(2 * C + sb, LANES)] = b01 + z0
                    idx_v[pl.ds(3 * C + sb, LANES)] = b01 + z1
                    idx_v[pl.ds(4 * C + sb, LANES)] = b10 + z0
                    idx_v[pl.ds(5 * C + sb, LANES)] = b10 + z1
                    idx_v[pl.ds(6 * C + sb, LANES)] = b11 + z0
                    idx_v[pl.ds(7 * C + sb, LANES)] = b11 + z1
                    wt_v[pl.ds(0 * C + sb, LANES)] = xd
                    wt_v[pl.ds(1 * C + sb, LANES)] = yd
                    wt_v[pl.ds(2 * C + sb, LANES)] = zd
                    return carry2

                lax.fori_loop(0, C // LANES, vec_body, 0)

                cps = [
                    pltpu.async_copy(gridf_hbm.at[idx_v.at[pl.ds(k * C, C)]],
                                     val_v.at[pl.ds(k * C, C)], sem)
                    for k in range(8)
                ]
                for cp in cps:
                    cp.wait()

                def mix_body(j, carry2):
                    sb = j * LANES
                    dsj = pl.ds(sb, LANES)
                    v000 = val_v[pl.ds(0 * C + sb, LANES)]
                    v001 = val_v[pl.ds(1 * C + sb, LANES)]
                    v010 = val_v[pl.ds(2 * C + sb, LANES)]
                    v011 = val_v[pl.ds(3 * C + sb, LANES)]
                    v100 = val_v[pl.ds(4 * C + sb, LANES)]
                    v101 = val_v[pl.ds(5 * C + sb, LANES)]
                    v110 = val_v[pl.ds(6 * C + sb, LANES)]
                    v111 = val_v[pl.ds(7 * C + sb, LANES)]
                    xd = wt_v[pl.ds(0 * C + sb, LANES)]
                    yd = wt_v[pl.ds(1 * C + sb, LANES)]
                    zd = wt_v[pl.ds(2 * C + sb, LANES)]
                    c00 = v000 + (v100 - v000) * xd
                    c01 = v001 + (v101 - v001) * xd
                    c10 = v010 + (v110 - v010) * xd
                    c11 = v011 + (v111 - v011) * xd
                    c0 = c00 + (c10 - c00) * yd
                    c1 = c01 + (c11 - c01) * yd
                    out_v[dsj] = c0 + (c1 - c0) * zd
                    return carry2

                lax.fori_loop(0, C // LANES, mix_body, 0)
                pltpu.sync_copy(out_v, out_hbm.at[pl.ds(base, C)])

            return carry

        lax.fori_loop(0, niter, iter_body, 0)

    return sc_interp


def kernel(points, grid):
    npts = points.shape[0]
    nchunks = -(-npts // C)
    npad = nchunks * C
    pts = jnp.pad(points, ((0, npad - npts), (0, 0)))
    xyz = pts.reshape(nchunks, C, 3).transpose(0, 2, 1).reshape(-1)
    gridf = grid.reshape(-1)
    out = _make_sc_interp(npad, nchunks)(xyz, gridf)
    return out[:npts]
